# baseline (speedup 1.0000x reference)
import functools
import jax
import jax.numpy as jnp
from jax import lax
from jax.experimental import pallas as pl
from jax.experimental.pallas import tpu as pltpu
from jax.experimental.pallas import tpu_sc as plsc

N_CS = 10000; N_IN = 10000; E_CS = 160000; E_IN = 160000
D = 256; DE = 16; DG = 16; H = 256; B = 64

NC, NS, LANES = 2, 16, 16
CH = 128
CPS = E_CS // NC // CH
NACC = 10240
ZR = 128
EMACC = 64


def _sc_mesh():
    return plsc.VectorSubcoreMesh(core_axis_name="c", subcore_axis_name="s",
                                  num_cores=NC, num_subcores=NS)


HALF = N_CS // 2
HPAD = 5120
NCHUNK = E_CS // CH
_IOTA = lambda: lax.iota(jnp.int32, LANES)


def _seg_scatter_body(v1_hbm, dst_hbm, wgt_hbm, z1_hbm, out_n,
                      out_c, dstv, locv, wv, rows, acc, hist):
    c = lax.axis_index("c")
    s = lax.axis_index("s")
    for j in range(HPAD * LANES // 8192):
        pltpu.sync_copy(z1_hbm, acc.at[pl.ds(j * 8192, 8192)])
    pltpu.sync_copy(z1_hbm.at[pl.ds(0, HPAD)], hist)
    half0 = c * HALF
    lane0 = _IOTA() == 0
    vbase0 = s * (E_CS * LANES)

    def it_body(ch, carry):
        base = ch * CH
        pltpu.sync_copy(dst_hbm.at[pl.ds(base, CH)], dstv)
        pltpu.sync_copy(wgt_hbm.at[pl.ds(base, CH)], wv)
        pltpu.sync_copy(v1_hbm.at[pl.ds(vbase0 + base * LANES, CH * LANES)],
                        rows)
        for g in range(CH // LANES):
            sl = pl.ds(g * LANES, LANES)
            loc = dstv[sl] - half0
            ok = (loc >= 0) & (loc < HALF)
            locv[sl] = jnp.where(ok, loc, HALF)
        for p in range(CH):
            fp = jnp.full((LANES,), p, jnp.int32)
            spl = plsc.load_gather(locv, [fp])
            row = plsc.load_gather(rows, [fp * LANES + _IOTA()])
            wspl = plsc.load_gather(wv, [fp])
            plsc.addupdate_scatter(acc, [spl * LANES + _IOTA()], row * wspl)
            plsc.addupdate_scatter(hist, [spl], wspl, mask=lane0)
        return carry

    lax.fori_loop(0, NCHUNK, it_body, 0)
    pltpu.sync_copy(acc,
                    out_n.at[pl.ds((c * NS + s) * (HPAD * LANES),
                                   HPAD * LANES)])

    @pl.when(s == 0)
    def _():
        pltpu.sync_copy(hist, out_c.at[pl.ds(c * HPAD, HPAD)])


def _seg_scatter(vals, dst, wgt):
    z1 = jnp.zeros((8192,), jnp.float32)
    f = pl.kernel(
        _seg_scatter_body,
        out_type=[
            jax.ShapeDtypeStruct((NC * NS * HPAD * LANES,), jnp.float32),
            jax.ShapeDtypeStruct((NC * HPAD,), jnp.float32),
        ],
        mesh=_sc_mesh(),
        compiler_params=pltpu.CompilerParams(needs_layout_passes=False),
        scratch_types=[
            pltpu.VMEM((CH,), jnp.int32),
            pltpu.VMEM((CH,), jnp.int32),
            pltpu.VMEM((CH,), jnp.float32),
            pltpu.VMEM((CH * LANES,), jnp.float32),
            pltpu.VMEM((HPAD * LANES,), jnp.float32),
            pltpu.VMEM((HPAD,), jnp.float32),
        ],
    )
    v1 = vals.reshape(-1, LANES, LANES).transpose(1, 0, 2).reshape(-1)
    out_n, out_c = f(v1, dst, wgt, z1)
    out_n = (out_n.reshape(NC, NS, HPAD, LANES)
             .transpose(0, 2, 1, 3).reshape(NC * HPAD, D))
    n_sum = jnp.concatenate([out_n[:HALF], out_n[HPAD:HPAD + HALF]], 0)
    n_cnt = jnp.concatenate([out_c[:HALF], out_c[HPAD:HPAD + HALF]], 0)
    return n_sum, n_cnt


def _bmean_kernel(ids_ref, vals_ref, o_ref, c_ref):
    k = pl.program_id(0)
    ids = ids_ref[0, 0]
    oh = (lax.broadcasted_iota(jnp.int32, (B, ids.shape[0]), 0)
          == ids[None, :].astype(jnp.int32)).astype(jnp.float32)

    @pl.when(k == 0)
    def _():
        o_ref[...] = jnp.zeros_like(o_ref)
        c_ref[...] = jnp.zeros_like(c_ref)

    o_ref[...] += jnp.dot(oh, vals_ref[...], preferred_element_type=jnp.float32)
    c_ref[...] += jnp.sum(oh, axis=1, keepdims=True)


def _batch_mean(vals, ids, bk=2000):
    m, n = vals.shape
    grid = (m // bk,)
    sm, cn = pl.pallas_call(
        _bmean_kernel,
        grid=grid,
        in_specs=[pl.BlockSpec((1, 1, bk), lambda k: (k, 0, 0)),
                  pl.BlockSpec((bk, n), lambda k: (k, 0))],
        out_specs=[pl.BlockSpec((B, n), lambda k: (0, 0)),
                   pl.BlockSpec((B, 1), lambda k: (0, 0))],
        out_shape=[jax.ShapeDtypeStruct((B, n), jnp.float32),
                   jax.ShapeDtypeStruct((B, 1), jnp.float32)],
    )(ids.astype(jnp.float32).reshape(m // bk, 1, bk), vals)
    return sm / jnp.maximum(cn, 1.0)


def _expand_kernel(ids_ref, tab_ref, o_ref):
    ids = ids_ref[0, 0]
    oh = (lax.broadcasted_iota(jnp.int32, (ids.shape[0], B), 1)
          == ids[:, None].astype(jnp.int32)).astype(jnp.float32)
    o_ref[...] = jnp.dot(oh, tab_ref[...], preferred_element_type=jnp.float32)


def _expand_rows(table, ids, bk=2000):
    m = ids.shape[0]
    n = table.shape[1]
    return pl.pallas_call(
        _expand_kernel,
        grid=(m // bk,),
        in_specs=[pl.BlockSpec((1, 1, bk), lambda k: (k, 0, 0)),
                  pl.BlockSpec((B, n), lambda k: (0, 0))],
        out_specs=pl.BlockSpec((bk, n), lambda k: (k, 0)),
        out_shape=jax.ShapeDtypeStruct((m, n), jnp.float32),
    )(ids.astype(jnp.float32).reshape(m // bk, 1, bk), table)


def _mm_kernel(x_ref, w_ref, o_ref):
    o_ref[...] = jnp.dot(x_ref[...], w_ref[...],
                         preferred_element_type=jnp.float32)


def _mm(x, w, bm=512):
    m, k = x.shape
    k2, n = w.shape
    grid = (pl.cdiv(m, bm),)
    return pl.pallas_call(
        _mm_kernel,
        grid=grid,
        in_specs=[pl.BlockSpec((bm, k), lambda i: (i, 0)),
                  pl.BlockSpec((k, n), lambda i: (0, 0))],
        out_specs=pl.BlockSpec((bm, n), lambda i: (i, 0)),
        out_shape=jax.ShapeDtypeStruct((m, n), jnp.float32),
    )(x, w)


def _seg_mean(vals, ids, num):
    sm = jax.ops.segment_sum(vals, ids, num_segments=num)
    cn = jax.ops.segment_sum(jnp.ones((vals.shape[0],), vals.dtype), ids,
                             num_segments=num)
    return sm / jnp.maximum(cn, 1.0)[:, None]


def kernel(cs_x, in_x, cs_edge_index, in_edge_sources, in_edge_targets,
           cs_edge_attr, in_edge_attr, global_attr, cs_node_batch,
           in_node_batch, W_gat, a_att, W_gat_out, Wp1, Wp2, We1, We2,
           Wn1, Wn2, Wg1, Wg2, Wr, Wm1, bm1, Wm2, bm2):
    h = _mm(in_x, W_gat)
    a1 = a_att[:H, 0]; a2 = a_att[H:2 * H, 0]; a3 = a_att[2 * H:, 0]
    s1 = h @ a1
    s2 = h @ a2
    e3 = in_edge_attr @ a3
    logits = s1[in_edge_sources] + s2[in_edge_targets] + e3
    logits = jnp.where(logits >= 0, logits, 0.2 * logits)
    ex = jnp.exp(logits)
    hs = h[in_edge_sources]
    num, den = _seg_scatter(hs, in_edge_targets, ex)
    agg = num / (den[:, None] + 1e-16)
    node_in = jax.nn.relu(_mm(agg, W_gat_out))
    pooled_in = _batch_mean(node_in, in_node_batch)
    in_out = jax.nn.relu(jnp.concatenate([pooled_in, global_attr], 1) @ Wp1)
    in_out = jax.nn.relu(in_out @ Wp2)
    src = cs_edge_index[0]; dst = cs_edge_index[1]
    eb = cs_node_batch[src]
    p_src = _mm(cs_x, We1[:D])
    p_dst = _mm(cs_x, We1[D:2 * D])
    p_e = _mm(cs_edge_attr, We1[2 * D:2 * D + DE])
    p_g = global_attr @ We1[2 * D + DE:]
    e_pre = jax.nn.relu(p_src[src] + p_dst[dst] + p_e + _expand_rows(p_g, eb))
    e_h = jax.nn.relu(_mm(e_pre, We2))
    n_sum, n_cnt = _seg_scatter(e_h, dst, jnp.ones((E_CS,), jnp.float32))
    e2n = n_sum / jnp.maximum(n_cnt, 1.0)[:, None]
    n_pre = jax.nn.relu(_mm(cs_x, Wn1[:D]) + _mm(e2n, Wn1[D:2 * D])
                        + _expand_rows(global_attr @ Wn1[2 * D:],
                                       cs_node_batch))
    n_h = jax.nn.relu(_mm(n_pre, Wn2))
    node_mean = _batch_mean(n_h, cs_node_batch)
    edge_mean = _batch_mean(e_h, eb)
    gcat = jnp.concatenate([node_mean, edge_mean, global_attr], 1)
    g_h = jax.nn.relu(jax.nn.relu(gcat @ Wg1) @ Wg2)
    cs_out = jax.nn.relu(jnp.concatenate([node_mean, edge_mean, g_h], 1) @ Wr)
    merged = jnp.concatenate([in_out, cs_out], 1)
    hm = jax.nn.relu(merged @ Wm1 + bm1)
    final = hm @ Wm2 + bm2
    return final.reshape(-1)


# speedup vs baseline: 1.6597x; 1.6597x over previous
import functools
import jax
import jax.numpy as jnp
from jax import lax
from jax.experimental import pallas as pl
from jax.experimental.pallas import tpu as pltpu
from jax.experimental.pallas import tpu_sc as plsc

N_CS = 10000; N_IN = 10000; E_CS = 160000; E_IN = 160000
D = 256; DE = 16; DG = 16; H = 256; B = 64

NC, NS, LANES = 2, 16, 16
CH = 256
CPS = E_CS // NC // CH
NACC = 10240
ZR = 128
EMACC = 64


def _sc_mesh():
    return plsc.VectorSubcoreMesh(core_axis_name="c", subcore_axis_name="s",
                                  num_cores=NC, num_subcores=NS)


HALF = N_CS // 2
HPAD = 5120
NCHUNK = E_CS // CH
_IOTA = lambda: lax.iota(jnp.int32, LANES)


def _seg_scatter_body(v1_hbm, dst_hbm, wgt_hbm, z1_hbm, out_n, out_c,
                      dstv0, dstv1, locv, wv0, wv1, rows0, rows1,
                      acc, hist, sem0, sem1):
    c = lax.axis_index("c")
    s = lax.axis_index("s")
    for j in range(HPAD * LANES // 8192):
        pltpu.sync_copy(z1_hbm, acc.at[pl.ds(j * 8192, 8192)])
    pltpu.sync_copy(z1_hbm.at[pl.ds(0, HPAD)], hist)
    half0 = c * HALF
    lane0 = _IOTA() == 0
    lall = _IOTA() >= 0
    vbase0 = s * (E_CS * LANES)
    dbufs = (dstv0, dstv1)
    wbufs = (wv0, wv1)
    rbufs = (rows0, rows1)
    sems = (sem0, sem1)

    def issue(ch, b):
        base = ch * CH
        pltpu.async_copy(dst_hbm.at[pl.ds(base, CH)], dbufs[b], sems[b])
        pltpu.async_copy(wgt_hbm.at[pl.ds(base, CH)], wbufs[b], sems[b])
        pltpu.async_copy(v1_hbm.at[pl.ds(vbase0 + base * LANES, CH * LANES)],
                         rbufs[b], sems[b])

    def drain(ch, b):
        base = ch * CH
        pltpu.make_async_copy(dst_hbm.at[pl.ds(base, CH)], dbufs[b],
                              sems[b]).wait()
        pltpu.make_async_copy(wgt_hbm.at[pl.ds(base, CH)], wbufs[b],
                              sems[b]).wait()
        pltpu.make_async_copy(v1_hbm.at[pl.ds(vbase0 + base * LANES,
                                              CH * LANES)],
                              rbufs[b], sems[b]).wait()

    issue(0, 0)
    issue(1, 1)

    def it_body(i, carry):
        for b in range(2):
            ch = 2 * i + b

            @pl.when(ch < NCHUNK)
            def _():
                drain(ch, b)
                dstv, wv, rows = dbufs[b], wbufs[b], rbufs[b]
                for g in range(CH // LANES):
                    sl = pl.ds(g * LANES, LANES)
                    loc = dstv[sl] - half0
                    ok = (loc >= 0) & (loc < HALF)
                    locv[sl] = jnp.where(ok, loc, HALF)
                for p in range(CH):
                    fp = jnp.full((LANES,), p, jnp.int32)
                    spl = plsc.load_gather(locv, [fp])
                    row = plsc.load_gather(rows, [fp * LANES + _IOTA()])
                    wspl = plsc.load_gather(wv, [fp])
                    plsc.addupdate_scatter(acc, [spl * LANES + _IOTA()],
                                           row * wspl, mask=lall)
                    plsc.addupdate_scatter(hist, [spl], wspl, mask=lane0)

                @pl.when(ch + 2 < NCHUNK)
                def _():
                    issue(ch + 2, b)

        return carry

    lax.fori_loop(0, (NCHUNK + 1) // 2, it_body, 0)
    pltpu.sync_copy(acc,
                    out_n.at[pl.ds((c * NS + s) * (HPAD * LANES),
                                   HPAD * LANES)])

    @pl.when(s == 0)
    def _():
        pltpu.sync_copy(hist, out_c.at[pl.ds(c * HPAD, HPAD)])


def _seg_scatter(vals, dst, wgt):
    z1 = jnp.zeros((8192,), jnp.float32)
    f = pl.kernel(
        _seg_scatter_body,
        out_type=[
            jax.ShapeDtypeStruct((NC * NS * HPAD * LANES,), jnp.float32),
            jax.ShapeDtypeStruct((NC * HPAD,), jnp.float32),
        ],
        mesh=_sc_mesh(),
        compiler_params=pltpu.CompilerParams(needs_layout_passes=False),
        scratch_types=[
            pltpu.VMEM((CH,), jnp.int32),
            pltpu.VMEM((CH,), jnp.int32),
            pltpu.VMEM((CH,), jnp.int32),
            pltpu.VMEM((CH,), jnp.float32),
            pltpu.VMEM((CH,), jnp.float32),
            pltpu.VMEM((CH * LANES,), jnp.float32),
            pltpu.VMEM((CH * LANES,), jnp.float32),
            pltpu.VMEM((HPAD * LANES,), jnp.float32),
            pltpu.VMEM((HPAD,), jnp.float32),
            pltpu.SemaphoreType.DMA,
            pltpu.SemaphoreType.DMA,
        ],
    )
    v1 = vals.reshape(-1, LANES, LANES).transpose(1, 0, 2).reshape(-1)
    out_n, out_c = f(v1, dst, wgt, z1)
    out_n = (out_n.reshape(NC, NS, HPAD, LANES)
             .transpose(0, 2, 1, 3).reshape(NC * HPAD, D))
    n_sum = jnp.concatenate([out_n[:HALF], out_n[HPAD:HPAD + HALF]], 0)
    n_cnt = jnp.concatenate([out_c[:HALF], out_c[HPAD:HPAD + HALF]], 0)
    return n_sum, n_cnt


def _bmean_kernel(ids_ref, vals_ref, o_ref, c_ref):
    k = pl.program_id(0)
    ids = ids_ref[0, 0]
    oh = (lax.broadcasted_iota(jnp.int32, (B, ids.shape[0]), 0)
          == ids[None, :].astype(jnp.int32)).astype(jnp.float32)

    @pl.when(k == 0)
    def _():
        o_ref[...] = jnp.zeros_like(o_ref)
        c_ref[...] = jnp.zeros_like(c_ref)

    o_ref[...] += jnp.dot(oh, vals_ref[...], preferred_element_type=jnp.float32,
                          precision=lax.Precision.HIGHEST)
    c_ref[...] += jnp.sum(oh, axis=1, keepdims=True)


def _batch_mean(vals, ids, bk=2000):
    m, n = vals.shape
    grid = (m // bk,)
    sm, cn = pl.pallas_call(
        _bmean_kernel,
        grid=grid,
        in_specs=[pl.BlockSpec((1, 1, bk), lambda k: (k, 0, 0)),
                  pl.BlockSpec((bk, n), lambda k: (k, 0))],
        out_specs=[pl.BlockSpec((B, n), lambda k: (0, 0)),
                   pl.BlockSpec((B, 1), lambda k: (0, 0))],
        out_shape=[jax.ShapeDtypeStruct((B, n), jnp.float32),
                   jax.ShapeDtypeStruct((B, 1), jnp.float32)],
    )(ids.astype(jnp.float32).reshape(m // bk, 1, bk), vals)
    return sm / jnp.maximum(cn, 1.0)


def _expand_kernel(ids_ref, tab_ref, o_ref):
    ids = ids_ref[0, 0]
    oh = (lax.broadcasted_iota(jnp.int32, (ids.shape[0], B), 1)
          == ids[:, None].astype(jnp.int32)).astype(jnp.float32)
    o_ref[...] = jnp.dot(oh, tab_ref[...], preferred_element_type=jnp.float32,
                         precision=lax.Precision.HIGHEST)


def _expand_rows(table, ids, bk=2000):
    m = ids.shape[0]
    n = table.shape[1]
    return pl.pallas_call(
        _expand_kernel,
        grid=(m // bk,),
        in_specs=[pl.BlockSpec((1, 1, bk), lambda k: (k, 0, 0)),
                  pl.BlockSpec((B, n), lambda k: (0, 0))],
        out_specs=pl.BlockSpec((bk, n), lambda k: (k, 0)),
        out_shape=jax.ShapeDtypeStruct((m, n), jnp.float32),
    )(ids.astype(jnp.float32).reshape(m // bk, 1, bk), table)


def _mm_kernel(x_ref, w_ref, o_ref):
    o_ref[...] = jnp.dot(x_ref[...], w_ref[...],
                         preferred_element_type=jnp.float32,
                         precision=lax.Precision.HIGHEST)


def _mm(x, w, bm=512):
    m, k = x.shape
    k2, n = w.shape
    grid = (pl.cdiv(m, bm),)
    return pl.pallas_call(
        _mm_kernel,
        grid=grid,
        in_specs=[pl.BlockSpec((bm, k), lambda i: (i, 0)),
                  pl.BlockSpec((k, n), lambda i: (0, 0))],
        out_specs=pl.BlockSpec((bm, n), lambda i: (i, 0)),
        out_shape=jax.ShapeDtypeStruct((m, n), jnp.float32),
    )(x, w)


def _seg_mean(vals, ids, num):
    sm = jax.ops.segment_sum(vals, ids, num_segments=num)
    cn = jax.ops.segment_sum(jnp.ones((vals.shape[0],), vals.dtype), ids,
                             num_segments=num)
    return sm / jnp.maximum(cn, 1.0)[:, None]


def kernel(cs_x, in_x, cs_edge_index, in_edge_sources, in_edge_targets,
           cs_edge_attr, in_edge_attr, global_attr, cs_node_batch,
           in_node_batch, W_gat, a_att, W_gat_out, Wp1, Wp2, We1, We2,
           Wn1, Wn2, Wg1, Wg2, Wr, Wm1, bm1, Wm2, bm2):
    h = _mm(in_x, W_gat)
    a1 = a_att[:H, 0]; a2 = a_att[H:2 * H, 0]; a3 = a_att[2 * H:, 0]
    s1 = h @ a1
    s2 = h @ a2
    e3 = in_edge_attr @ a3
    logits = s1[in_edge_sources] + s2[in_edge_targets] + e3
    logits = jnp.where(logits >= 0, logits, 0.2 * logits)
    ex = jnp.exp(logits)
    hs = h[in_edge_sources]
    num = jax.ops.segment_sum(hs * ex[:, None], in_edge_targets,
                              num_segments=N_IN)
    den = jax.ops.segment_sum(ex, in_edge_targets, num_segments=N_IN)
    agg = num / (den[:, None] + 1e-16)
    node_in = jax.nn.relu(_mm(agg, W_gat_out))
    pooled_in = _batch_mean(node_in, in_node_batch)
    in_out = jax.nn.relu(jnp.concatenate([pooled_in, global_attr], 1) @ Wp1)
    in_out = jax.nn.relu(in_out @ Wp2)
    src = cs_edge_index[0]; dst = cs_edge_index[1]
    eb = cs_node_batch[src]
    p_src = _mm(cs_x, We1[:D])
    p_dst = _mm(cs_x, We1[D:2 * D])
    p_e = _mm(cs_edge_attr, We1[2 * D:2 * D + DE])
    p_g = global_attr @ We1[2 * D + DE:]
    e_pre = jax.nn.relu(p_src[src] + p_dst[dst] + p_e + _expand_rows(p_g, eb))
    e_h = jax.nn.relu(_mm(e_pre, We2))
    n_sum = jax.ops.segment_sum(e_h, dst, num_segments=N_CS)
    n_cnt = jax.ops.segment_sum(jnp.ones((E_CS,), jnp.float32), dst,
                                num_segments=N_CS)
    e2n = n_sum / jnp.maximum(n_cnt, 1.0)[:, None]
    n_pre = jax.nn.relu(_mm(cs_x, Wn1[:D]) + _mm(e2n, Wn1[D:2 * D])
                        + _expand_rows(global_attr @ Wn1[2 * D:],
                                       cs_node_batch))
    n_h = jax.nn.relu(_mm(n_pre, Wn2))
    node_mean = _batch_mean(n_h, cs_node_batch)
    edge_mean = _batch_mean(e_h, eb)
    gcat = jnp.concatenate([node_mean, edge_mean, global_attr], 1)
    g_h = jax.nn.relu(jax.nn.relu(gcat @ Wg1) @ Wg2)
    cs_out = jax.nn.relu(jnp.concatenate([node_mean, edge_mean, g_h], 1) @ Wr)
    merged = jnp.concatenate([in_out, cs_out], 1)
    hm = jax.nn.relu(merged @ Wm1 + bm1)
    final = hm @ Wm2 + bm2
    return final.reshape(-1)
